# baseline (speedup 1.0000x reference)
import functools

import jax
import jax.numpy as jnp
from jax import lax
from jax.experimental import pallas as pl
from jax.experimental.pallas import tpu as pltpu
from jax.experimental.pallas import tpu_sc as plsc

N = 10000
E = 320000
D = 128
B = 16

CH = 128
PER_TILE = E // 16
NFULL = PER_TILE // CH
REM = PER_TILE - NFULL * CH
NPAD = 10240
ROWS_T = NPAD // 16

BLK = 2000
NB = N // BLK

_S = jax.ShapeDtypeStruct


@functools.lru_cache(maxsize=None)
def _get_sc_deg():
    mesh = plsc.VectorSubcoreMesh(core_axis_name="c", subcore_axis_name="s")
    return functools.partial(
        pl.kernel,
        out_type=_S((2 * NPAD,), jnp.float32),
        mesh=mesh,
        scratch_types=[
            pltpu.VMEM((CH,), jnp.int32),
            pltpu.VMEM((REM,), jnp.int32),
            pltpu.VMEM((CH,), jnp.float32),
            pltpu.VMEM((640,), jnp.float32),
            pltpu.VMEM_SHARED((NPAD,), jnp.float32),
        ],
    )(_sc_deg_body)


def _sc_deg(dsts):
    return _get_sc_deg()(dsts)


def _sc_deg_body(dsts_hbm, deg_hbm, didx, didx_t, ones, zbuf, acc):
    c = lax.axis_index("c")
    s = lax.axis_index("s")
    for j in range(40):
        zbuf[pl.ds(j * 16, 16)] = jnp.zeros((16,), jnp.float32)
    for j in range(8):
        ones[pl.ds(j * 16, 16)] = jnp.ones((16,), jnp.float32)
    pltpu.sync_copy(zbuf, acc.at[pl.ds(s * 640, 640)])
    plsc.subcore_barrier()

    ebase = c * E + s * PER_TILE

    def chunk(i, _):
        base = pl.multiple_of(ebase + i * CH, 8)
        pltpu.sync_copy(dsts_hbm.at[pl.ds(base, CH)], didx)
        pltpu.sync_copy(ones, acc.at[didx], add=True)
        return 0

    lax.fori_loop(0, NFULL, chunk, 0)
    tbase = pl.multiple_of(ebase + NFULL * CH, 8)
    pltpu.sync_copy(dsts_hbm.at[pl.ds(tbase, REM)], didx_t)
    pltpu.sync_copy(ones.at[pl.ds(0, REM)], acc.at[didx_t], add=True)

    plsc.subcore_barrier()
    pltpu.sync_copy(acc.at[pl.ds(s * 640, 640)],
                    deg_hbm.at[pl.ds(c * NPAD + s * 640, 640)])


@functools.lru_cache(maxsize=None)
def _get_sc_msg():
    mesh = plsc.VectorSubcoreMesh(core_axis_name="c", subcore_axis_name="s")
    return functools.partial(
        pl.kernel,
        out_type=_S((2 * NPAD, D), jnp.float32),
        mesh=mesh,
        scratch_types=[
            pltpu.VMEM((CH,), jnp.int32),
            pltpu.VMEM((CH,), jnp.int32),
            pltpu.VMEM((CH, D), jnp.float32),
            pltpu.VMEM((REM,), jnp.int32),
            pltpu.VMEM((REM,), jnp.int32),
            pltpu.VMEM((REM, D), jnp.float32),
            pltpu.VMEM((32, D), jnp.float32),
            pltpu.VMEM_SHARED((NPAD, D), jnp.float32),
            pltpu.SemaphoreType.DMA,
        ],
    )(_sc_msg_body)


def _sc_msg(z_flat, srcs, dsts):
    return _get_sc_msg()(z_flat, srcs, dsts)


def _sc_msg_body(z_hbm, srcs_hbm, dsts_hbm, seg_hbm,
                 sidx, didx, rows, sidx_t, didx_t, rows_t, zbuf, acc, gsem):
    c = lax.axis_index("c")
    s = lax.axis_index("s")
    for i in range(32):
        for j in range(8):
            zbuf[i, pl.ds(j * 16, 16)] = jnp.zeros((16,), jnp.float32)
    rbase = s * ROWS_T
    for k in range(20):
        pltpu.sync_copy(zbuf, acc.at[pl.ds(rbase + k * 32, 32)])
    plsc.subcore_barrier()

    ebase = c * E + s * PER_TILE

    def chunk(i, _):
        base = pl.multiple_of(ebase + i * CH, 8)
        pltpu.sync_copy(srcs_hbm.at[pl.ds(base, CH)], sidx)
        pltpu.sync_copy(dsts_hbm.at[pl.ds(base, CH)], didx)
        pltpu.async_copy(z_hbm.at[sidx], rows, gsem).wait()
        pltpu.sync_copy(rows, acc.at[didx], add=True)
        return 0

    lax.fori_loop(0, NFULL, chunk, 0)
    tbase = pl.multiple_of(ebase + NFULL * CH, 8)
    pltpu.sync_copy(srcs_hbm.at[pl.ds(tbase, REM)], sidx_t)
    pltpu.sync_copy(dsts_hbm.at[pl.ds(tbase, REM)], didx_t)
    pltpu.async_copy(z_hbm.at[sidx_t], rows_t, gsem).wait()
    pltpu.sync_copy(rows_t, acc.at[didx_t], add=True)

    plsc.subcore_barrier()
    pltpu.sync_copy(acc.at[pl.ds(rbase, ROWS_T)],
                    seg_hbm.at[pl.ds(c * NPAD + rbase, ROWS_T)])


def _dot(a, b):
    return jnp.dot(a, b, preferred_element_type=jnp.float32)


def _lin0_body(x_ref, w_ref, b_ref, o_ref):
    o_ref[0] = jnp.maximum(_dot(x_ref[0], w_ref[0]) + b_ref[0], 0.0)


def _lin0(x, w0t, b0):
    return pl.pallas_call(
        _lin0_body,
        grid=(2, NB),
        in_specs=[
            pl.BlockSpec((1, BLK, D), lambda c, i: (c, i, 0)),
            pl.BlockSpec((1, D, D), lambda c, i: (c, 0, 0)),
            pl.BlockSpec((1, 1, D), lambda c, i: (c, 0, 0)),
        ],
        out_specs=pl.BlockSpec((1, BLK, D), lambda c, i: (c, i, 0)),
        out_shape=_S((2, N, D), jnp.float32),
    )(x, w0t, b0)


def _zprep_body(h_ref, w_ref, deg_ref, o_ref):
    dinv = lax.rsqrt(deg_ref[0] + 1.0)
    o_ref[0] = _dot(h_ref[0], w_ref[0]) * dinv


def _zprep(h, convw, deg_col):
    return pl.pallas_call(
        _zprep_body,
        grid=(2, NB),
        in_specs=[
            pl.BlockSpec((1, BLK, D), lambda c, i: (c, i, 0)),
            pl.BlockSpec((1, D, D), lambda c, i: (c, 0, 0)),
            pl.BlockSpec((1, BLK, 1), lambda c, i: (c, i, 0)),
        ],
        out_specs=pl.BlockSpec((1, BLK, D), lambda c, i: (c, i, 0)),
        out_shape=_S((2, N, D), jnp.float32),
    )(h, convw, deg_col)


def _gru_body(seg_ref, z_ref, deg_ref, cb_ref, h_ref, wih_ref, whh_ref,
              bih_ref, bhh_ref, o_ref):
    dinv = lax.rsqrt(deg_ref[0] + 1.0)
    m = jnp.maximum((seg_ref[0] + z_ref[0]) * dinv + cb_ref[0], 0.0)
    h = h_ref[0]
    gi = _dot(m, wih_ref[0]) + bih_ref[0]
    gh = _dot(h, whh_ref[0]) + bhh_ref[0]
    i_r, i_z, i_n = jnp.split(gi, 3, axis=-1)
    h_r, h_z, h_n = jnp.split(gh, 3, axis=-1)
    r = jax.nn.sigmoid(i_r + h_r)
    zz = jax.nn.sigmoid(i_z + h_z)
    nn_ = jnp.tanh(i_n + r * h_n)
    o_ref[0] = (1.0 - zz) * nn_ + zz * h


def _gru(segp, z, deg_col, conv_b, h, wiht, whht, bih, bhh):
    return pl.pallas_call(
        _gru_body,
        grid=(2, NB),
        in_specs=[
            pl.BlockSpec((1, BLK, D), lambda c, i: (c, i, 0)),
            pl.BlockSpec((1, BLK, D), lambda c, i: (c, i, 0)),
            pl.BlockSpec((1, BLK, 1), lambda c, i: (c, i, 0)),
            pl.BlockSpec((1, 1, D), lambda c, i: (c, 0, 0)),
            pl.BlockSpec((1, BLK, D), lambda c, i: (c, i, 0)),
            pl.BlockSpec((1, D, 3 * D), lambda c, i: (c, 0, 0)),
            pl.BlockSpec((1, D, 3 * D), lambda c, i: (c, 0, 0)),
            pl.BlockSpec((1, 1, 3 * D), lambda c, i: (c, 0, 0)),
            pl.BlockSpec((1, 1, 3 * D), lambda c, i: (c, 0, 0)),
        ],
        out_specs=pl.BlockSpec((1, BLK, D), lambda c, i: (c, i, 0)),
        out_shape=_S((2, N, D), jnp.float32),
    )(segp, z, deg_col, conv_b, h, wiht, whht, bih, bhh)


_S2S_CHUNK = 1000
_S2S_NC = N // _S2S_CHUNK


def _set2set_body(x_ref, batch_ref, wih_ref, whh_ref, bih_ref, bhh_ref, o_ref,
                  e_scr, ex_scr):
    wih = wih_ref[0]
    whh = whh_ref[0]
    bih = bih_ref[0]
    bhh = bhh_ref[0]
    iota16 = lax.broadcasted_iota(jnp.int32, (1, B), 1)

    h = jnp.zeros((B, D), jnp.float32)
    cc = jnp.zeros((B, D), jnp.float32)
    q_star = jnp.zeros((B, 2 * D), jnp.float32)

    C = _S2S_CHUNK
    for _ in range(3):
        gates = _dot(q_star, wih) + bih + _dot(h, whh) + bhh
        ii, ff, gg, oo = jnp.split(gates, 4, axis=-1)
        cc = jax.nn.sigmoid(ff) * cc + jax.nn.sigmoid(ii) * jnp.tanh(gg)
        h = jax.nn.sigmoid(oo) * jnp.tanh(cc)
        q = h

        em16 = jnp.full((1, B), -jnp.inf, jnp.float32)
        for ci in range(_S2S_NC):
            x_c = x_ref[0, pl.ds(ci * C, C), :]
            b_c = batch_ref[0, 0, pl.ds(ci * C, C)].reshape(C, 1)
            qsel = jnp.zeros((C, D), jnp.float32)
            for g in range(B):
                q_row = lax.slice(q, (g, 0), (g + 1, D))
                mask = jnp.where(b_c == g, 1.0, 0.0)
                qsel = qsel + mask * jnp.broadcast_to(q_row, (C, D))
            e_c = jnp.sum(x_c * qsel, axis=1, keepdims=True)
            e_scr[pl.ds(ci * C, C), :] = e_c
            oh_c = jnp.where(b_c == iota16, 1.0, 0.0)
            masked = jnp.where(oh_c > 0.5, jnp.broadcast_to(e_c, (C, B)),
                               -jnp.inf)
            em16 = jnp.maximum(em16, jnp.max(masked, axis=0, keepdims=True))
        em16 = jnp.where(jnp.isfinite(em16), em16, 0.0)

        den16 = jnp.zeros((1, B), jnp.float32)
        for ci in range(_S2S_NC):
            b_c = batch_ref[0, 0, pl.ds(ci * C, C)].reshape(C, 1)
            oh_c = jnp.where(b_c == iota16, 1.0, 0.0)
            emsel = jnp.sum(oh_c * em16, axis=1, keepdims=True)
            ex_c = jnp.exp(e_scr[pl.ds(ci * C, C), :] - emsel)
            ex_scr[pl.ds(ci * C, C), :] = ex_c
            den16 = den16 + jnp.sum(oh_c * ex_c, axis=0, keepdims=True)

        r16 = jnp.zeros((B, D), jnp.float32)
        for ci in range(_S2S_NC):
            x_c = x_ref[0, pl.ds(ci * C, C), :]
            b_c = batch_ref[0, 0, pl.ds(ci * C, C)].reshape(C, 1)
            oh_c = jnp.where(b_c == iota16, 1.0, 0.0)
            densel = jnp.sum(oh_c * den16, axis=1, keepdims=True)
            a_c = ex_scr[pl.ds(ci * C, C), :] / (densel + 1e-16)
            ax = x_c * a_c
            rs = []
            for g in range(B):
                rg = jnp.sum(jnp.where(b_c == g, ax, 0.0), axis=0,
                             keepdims=True)
                rs.append(rg)
            r16 = r16 + jnp.concatenate(rs, axis=0)

        q_star = jnp.concatenate([q, r16], axis=1)

    o_ref[0] = q_star


def _set2set(x, batch3, wiht, whht, bih, bhh):
    return pl.pallas_call(
        _set2set_body,
        grid=(2,),
        in_specs=[
            pl.BlockSpec((1, N, D), lambda c: (c, 0, 0)),
            pl.BlockSpec((1, 1, N), lambda c: (c, 0, 0)),
            pl.BlockSpec((1, 2 * D, 4 * D), lambda c: (c, 0, 0)),
            pl.BlockSpec((1, D, 4 * D), lambda c: (c, 0, 0)),
            pl.BlockSpec((1, 1, 4 * D), lambda c: (c, 0, 0)),
            pl.BlockSpec((1, 1, 4 * D), lambda c: (c, 0, 0)),
        ],
        out_specs=pl.BlockSpec((1, B, 2 * D), lambda c: (c, 0, 0)),
        out_shape=_S((2, B, 2 * D), jnp.float32),
        scratch_shapes=[
            pltpu.VMEM((N, 1), jnp.float32),
            pltpu.VMEM((N, 1), jnp.float32),
        ],
    )(x, batch3, wiht, whht, bih, bhh)


def _fusion_body(cat_ref, w1t_ref, b1_ref, w2t_ref, o_ref):
    hfc = jnp.maximum(_dot(cat_ref[...], w1t_ref[...]) + b1_ref[...], 0.0)
    o_ref[...] = _dot(hfc, w2t_ref[...])


def _fusion(cat, fc1_W, fc1_b, fc2_W, fc2_b):
    w2t = jnp.zeros((D, 8), jnp.float32).at[:, 0].set(fc2_W[0])
    out = pl.pallas_call(
        _fusion_body,
        out_shape=_S((B, 8), jnp.float32),
    )(cat, fc1_W.T, fc1_b.reshape(1, -1), w2t)
    return out[:, 0] + fc2_b[0]


def _encoder_pair(x, srcs, dsts, p1, p2):
    st = lambda k: jnp.stack([p1[k], p2[k]])
    stT = lambda k: jnp.stack([p1[k].T, p2[k].T])
    w0t = stT("lin0_W")
    b0 = st("lin0_b").reshape(2, 1, D)
    convw = st("conv_W")
    conv_b = st("conv_b").reshape(2, 1, D)
    wiht = stT("gru_Wih")
    whht = stT("gru_Whh")
    bih = st("gru_bih").reshape(2, 1, 3 * D)
    bhh = st("gru_bhh").reshape(2, 1, 3 * D)

    degp = _sc_deg(dsts)
    deg_col = degp.reshape(2, NPAD, 1)[:, :N, :]

    out = _lin0(x, w0t, b0)
    h = out
    for _ in range(3):
        z = _zprep(h, convw, deg_col)
        seg = _sc_msg(z.reshape(2 * N, D), srcs, dsts)
        segp = seg.reshape(2, NPAD, D)
        h = _gru(segp, z, deg_col, conv_b, h, wiht, whht, bih, bhh)
    return h


def kernel(x1, x2, edge_index1, edge_index2, x1_batch, x2_batch,
           e1_lin0_W, e1_lin0_b, e1_conv_W, e1_conv_b,
           e1_gru_Wih, e1_gru_Whh, e1_gru_bih, e1_gru_bhh,
           e1_lstm_Wih, e1_lstm_Whh, e1_lstm_bih, e1_lstm_bhh,
           e2_lin0_W, e2_lin0_b, e2_conv_W, e2_conv_b,
           e2_gru_Wih, e2_gru_Whh, e2_gru_bih, e2_gru_bhh,
           e2_lstm_Wih, e2_lstm_Whh, e2_lstm_bih, e2_lstm_bhh,
           fc1_W, fc1_b, fc2_W, fc2_b):
    p1 = dict(lin0_W=e1_lin0_W, lin0_b=e1_lin0_b, conv_W=e1_conv_W,
              conv_b=e1_conv_b, gru_Wih=e1_gru_Wih, gru_Whh=e1_gru_Whh,
              gru_bih=e1_gru_bih, gru_bhh=e1_gru_bhh)
    p2 = dict(lin0_W=e2_lin0_W, lin0_b=e2_lin0_b, conv_W=e2_conv_W,
              conv_b=e2_conv_b, gru_Wih=e2_gru_Wih, gru_Whh=e2_gru_Whh,
              gru_bih=e2_gru_bih, gru_bhh=e2_gru_bhh)

    x = jnp.stack([x1, x2])
    srcs = jnp.concatenate([edge_index1[0], edge_index2[0] + N])
    dsts = jnp.concatenate([edge_index1[1], edge_index2[1]])

    hfin = _encoder_pair(x, srcs, dsts, p1, p2)

    batch3 = jnp.stack([x1_batch, x2_batch]).reshape(2, 1, N)
    wiht_l = jnp.stack([e1_lstm_Wih.T, e2_lstm_Wih.T])
    whht_l = jnp.stack([e1_lstm_Whh.T, e2_lstm_Whh.T])
    bih_l = jnp.stack([e1_lstm_bih, e2_lstm_bih]).reshape(2, 1, 4 * D)
    bhh_l = jnp.stack([e1_lstm_bhh, e2_lstm_bhh]).reshape(2, 1, 4 * D)

    g = _set2set(hfin, batch3, wiht_l, whht_l, bih_l, bhh_l)
    cat = jnp.concatenate([g[0], g[1]], axis=1)
    return _fusion(cat, fc1_W, fc1_b, fc2_W, fc2_b)
